# baseline (speedup 1.0000x reference)
import functools

import jax
import jax.numpy as jnp
from jax import lax
from jax.experimental import pallas as pl
from jax.experimental.pallas import tpu as pltpu
from jax.experimental.pallas import tpu_sc as plsc

NUM_T = 100
B = 32
L = 512
LANES = 16
NC, NS = 2, 16
JCHUNKS = L // LANES
ROW_BYTES = L * 4


def _pair_bias_body(zs_hbm, emb_hbm, out_hbm, zs_v, tab_v, rtab_v, sem):
    b = lax.axis_index("s") * NC + lax.axis_index("c")

    pltpu.sync_copy(zs_hbm.at[b], zs_v)
    pltpu.sync_copy(emb_hbm, tab_v)

    zch = [zs_v[pl.ds(c * LANES, LANES)] for c in range(JCHUNKS)]

    @plsc.parallel_loop(0, NUM_T, unroll=8)
    def v_body(v):
        base = v * NUM_T
        for c in range(JCHUNKS):
            vals = plsc.load_gather(tab_v, [zch[c] + base])
            rtab_v[pl.ds(v * L + c * LANES, LANES)] = vals

    def group_copies(g):
        rows = zs_v[pl.ds(g * LANES, LANES)]
        for r in range(LANES):
            v = rows[r]
            yield pltpu.make_async_copy(
                rtab_v.at[pl.ds(v * L, L)],
                out_hbm.at[b, g * LANES + r],
                sem,
            )

    def issue_group(g):
        for cp in group_copies(g):
            cp.start()

    def wait_group(g):
        for cp in group_copies(g):
            cp.wait()

    issue_group(0)
    issue_group(1)

    def g_body(g, _):
        issue_group(g)
        wait_group(g - 2)
        return 0

    lax.fori_loop(2, L // LANES, g_body, 0)
    wait_group(L // LANES - 2)
    wait_group(L // LANES - 1)


@functools.partial(jax.jit, static_argnames=())
def kernel(zs, pair_emb):
    zs32 = zs.astype(jnp.int32)
    emb = pair_emb.reshape(NUM_T * NUM_T)

    mesh = plsc.VectorSubcoreMesh(core_axis_name="c", subcore_axis_name="s")
    run = pl.kernel(
        _pair_bias_body,
        out_type=jax.ShapeDtypeStruct((B, L, L), jnp.float32),
        mesh=mesh,
        compiler_params=pltpu.CompilerParams(needs_layout_passes=False),
        scratch_types=[
            pltpu.VMEM((L,), jnp.int32),
            pltpu.VMEM((NUM_T * NUM_T,), jnp.float32),
            pltpu.VMEM((NUM_T * L,), jnp.float32),
            pltpu.SemaphoreType.DMA,
        ],
    )
    return run(zs32, emb)


# speedup vs baseline: 1.0293x; 1.0293x over previous
import functools

import jax
import jax.numpy as jnp
from jax import lax
from jax.experimental import pallas as pl
from jax.experimental.pallas import tpu as pltpu
from jax.experimental.pallas import tpu_sc as plsc

NUM_T = 100
B = 32
L = 512
LANES = 16
NC, NS = 2, 16
JCHUNKS = L // LANES
ROW_BYTES = L * 4


def _pair_bias_body(zs_hbm, emb_hbm, out_hbm, zs_v, tab_v, rtab_v, sem):
    b = lax.axis_index("s") * NC + lax.axis_index("c")

    pltpu.sync_copy(zs_hbm.at[b], zs_v)
    pltpu.sync_copy(emb_hbm, tab_v)

    zch = [zs_v[pl.ds(c * LANES, LANES)] for c in range(JCHUNKS)]

    @plsc.parallel_loop(0, NUM_T, unroll=4)
    def v_body(v):
        base = v * NUM_T
        for c in range(JCHUNKS):
            vals = plsc.load_gather(tab_v, [zch[c] + base])
            rtab_v[pl.ds(v * L + c * LANES, LANES)] = vals

    def group_copies(g):
        rows = zs_v[pl.ds(g * LANES, LANES)]
        for r in range(LANES):
            v = rows[r]
            yield pltpu.make_async_copy(
                rtab_v.at[pl.ds(v * L, L)],
                out_hbm.at[b, g * LANES + r],
                sem,
            )

    def issue_group(g):
        for cp in group_copies(g):
            cp.start()

    def wait_group(g):
        del g
        for _ in range(LANES):
            pltpu.make_async_copy(
                rtab_v.at[pl.ds(0, L)], out_hbm.at[b, 0], sem
            ).wait()

    issue_group(0)
    issue_group(1)

    def g_body(g, _):
        issue_group(g)
        wait_group(g - 2)
        return 0

    lax.fori_loop(2, L // LANES, g_body, 0)
    wait_group(L // LANES - 2)
    wait_group(L // LANES - 1)


@functools.partial(jax.jit, static_argnames=())
def kernel(zs, pair_emb):
    zs32 = zs.astype(jnp.int32)
    emb = pair_emb.reshape(NUM_T * NUM_T)

    mesh = plsc.VectorSubcoreMesh(core_axis_name="c", subcore_axis_name="s")
    run = pl.kernel(
        _pair_bias_body,
        out_type=jax.ShapeDtypeStruct((B, L, L), jnp.float32),
        mesh=mesh,
        compiler_params=pltpu.CompilerParams(needs_layout_passes=False),
        scratch_types=[
            pltpu.VMEM((L,), jnp.int32),
            pltpu.VMEM((NUM_T * NUM_T,), jnp.float32),
            pltpu.VMEM((NUM_T * L,), jnp.float32),
            pltpu.SemaphoreType.DMA,
        ],
    )
    return run(zs32, emb)
